# Initial kernel scaffold; baseline (speedup 1.0000x reference)
import jax
import jax.numpy as jnp
from jax.experimental import pallas as pl


def kernel(x, ws_gate, ws_up, ws_down, w_router, w_gate, w_up, w_down):
    raise NotImplementedError("write your pallas kernel here")


# speedup vs baseline: 1.6414x; 1.6414x over previous
import functools

import jax
import jax.numpy as jnp
from jax.experimental import pallas as pl
from jax.experimental.pallas import tpu as pltpu

H = 1024
I = 512
E = 8
TOP_K = 2
NS = 1

BLK = 256


def _moe_dense_kernel(x_ref, wsg_ref, wsu_ref, wsd_ref, wr_ref,
                      wg_ref, wu_ref, wd_ref, out_ref):
    x32 = x_ref[...]
    xb = x32.astype(jnp.bfloat16)

    logits = jax.lax.dot_general(
        x32, wr_ref[...], (((1,), (1,)), ((), ())),
        preferred_element_type=jnp.float32)
    m = jnp.max(logits, axis=1, keepdims=True)
    p = jnp.exp(logits - m)
    p = p / jnp.sum(p, axis=1, keepdims=True)
    eids = jax.lax.broadcasted_iota(jnp.int32, (BLK, E), 1)
    w0 = jnp.max(p, axis=1, keepdims=True)
    i0 = jnp.argmax(p, axis=1, keepdims=True)
    p2 = jnp.where(eids == i0, -1.0, p)
    w1 = jnp.max(p2, axis=1, keepdims=True)
    i1 = jnp.argmax(p2, axis=1, keepdims=True)

    acc = x32
    for s in range(NS):
        g = jax.lax.dot_general(xb, wsg_ref[s], (((1,), (1,)), ((), ())),
                                preferred_element_type=jnp.float32)
        u = jax.lax.dot_general(xb, wsu_ref[s], (((1,), (1,)), ((), ())),
                                preferred_element_type=jnp.float32)
        h = (jax.nn.sigmoid(g) * u).astype(jnp.bfloat16)
        acc = acc + jax.lax.dot_general(h, wsd_ref[s], (((1,), (1,)), ((), ())),
                                        preferred_element_type=jnp.float32)

    for e in range(E):
        ge = jax.lax.dot_general(xb, wg_ref[e], (((1,), (1,)), ((), ())),
                                 preferred_element_type=jnp.float32)
        ue = jax.lax.dot_general(xb, wu_ref[e], (((1,), (1,)), ((), ())),
                                 preferred_element_type=jnp.float32)
        he = (jax.nn.sigmoid(ge) * ue).astype(jnp.bfloat16)
        de = jax.lax.dot_general(he, wd_ref[e], (((1,), (1,)), ((), ())),
                                 preferred_element_type=jnp.float32)
        scale = jnp.where(i0 == e, w0, 0.0) + jnp.where(i1 == e, w1, 0.0)
        acc = acc + (scale.astype(jnp.bfloat16)
                     * de.astype(jnp.bfloat16)).astype(jnp.float32)

    out_ref[...] = acc


@functools.partial(jax.jit, static_argnames=("interpret",))
def kernel(x, ws_gate, ws_up, ws_down, w_router, w_gate, w_up, w_down,
           interpret=False):
    B, S, Hx = x.shape
    flat = x.reshape(S * B, Hx)
    wsg = ws_gate.astype(jnp.bfloat16)
    wsu = ws_up.astype(jnp.bfloat16)
    wsd = ws_down.astype(jnp.bfloat16)
    wg = w_gate.astype(jnp.bfloat16)
    wu = w_up.astype(jnp.bfloat16)
    wd = w_down.astype(jnp.bfloat16)

    nblk = (S * B) // BLK
    full = lambda shape: pl.BlockSpec(shape, lambda i: (0,) * len(shape))
    out = pl.pallas_call(
        _moe_dense_kernel,
        grid=(nblk,),
        in_specs=[
            pl.BlockSpec((BLK, H), lambda i: (i, 0)),
            full((NS, I, H)), full((NS, I, H)), full((NS, H, I)),
            full((E, H)),
            full((E, I, H)), full((E, I, H)), full((E, H, I)),
        ],
        out_specs=pl.BlockSpec((BLK, H), lambda i: (i, 0)),
        out_shape=jax.ShapeDtypeStruct((S * B, Hx), jnp.float32),
        interpret=interpret,
    )(flat, wsg, wsu, wsd, w_router, wg, wu, wd)
    return out.reshape(B, S, Hx)
